# baseline (speedup 1.0000x reference)
import functools
import numpy as np
import jax
import jax.numpy as jnp
from jax import lax
from jax.experimental import pallas as pl
from jax.experimental.pallas import tpu as pltpu
from jax.experimental.pallas import tpu_sc as plsc

_MIN_EPS = 0.01
_MAX_EPS = 2.0
_N_EPS = 1000
_N_OM = 1000

_NC, _NS = 2, 16
_NW = _NC * _NS
_CN = 4096


_C1 = np.float32(1.0 / np.log(10.0))
_C2 = np.float32(434.588)
_RNE_MAGIC = np.float32(12582912.0)


def _tc_body(scale_ref, x_ref, y_ref, z_ref, om_ref, grow_ref):
    eps = scale_ref[...]
    fi = (jnp.log(eps) * _C1 + np.float32(2.0)) * _C2
    r = (fi + _RNE_MAGIC) - _RNE_MAGIC
    ei = jnp.clip(r.astype(jnp.int32), 0, _N_EPS - 1)
    grow_ref[...] = ei * _N_OM
    x = x_ref[...]
    y = y_ref[...]
    z = z_ref[...]
    om_ref[...] = jnp.sqrt((x * x + z * z) + y * y)


def _sc_main_body(om_hbm, grow_hbm, x_hbm, y_hbm, z_hbm, omg_hbm, tab_hbm,
                  ox_hbm, oy_hbm, oz_hbm,
                  ombuf, gbuf, xbuf, ybuf, zbuf, tbuf, ibuf,
                  g0b0, g0b1, g1b0, g1b1, f0b0, f0b1, f1b0, f1b1, obx, oby, obz, omg,
                  semi0, semi1, semg0, semg1, semo0, semo1,
                  nsub, inv_h, om0):
    sid = lax.axis_index("s")
    wid = sid * _NC + lax.axis_index("c")
    base = wid * (nsub * _CN)
    semi = (semi0, semi1)
    semg = (semg0, semg1)
    semo = (semo0, semo1)
    g0buf = (g0b0, g0b1)
    g1buf = (g1b0, g1b1)
    f0buf = (f0b0, f0b1)
    f1buf = (f1b0, f1b1)
    pltpu.sync_copy(omg_hbm, omg)

    def start_in(k, s):
        cs = pl.ds(base + k * _CN, _CN)
        return [
            pltpu.async_copy(om_hbm.at[cs], ombuf.at[s], semi[s]),
            pltpu.async_copy(grow_hbm.at[cs], gbuf.at[s], semi[s]),
            pltpu.async_copy(x_hbm.at[cs], xbuf.at[s], semi[s]),
            pltpu.async_copy(y_hbm.at[cs], ybuf.at[s], semi[s]),
            pltpu.async_copy(z_hbm.at[cs], zbuf.at[s], semi[s]),
        ]

    def start_out(k, s):
        cs = pl.ds(base + k * _CN, _CN)
        return [
            pltpu.async_copy(obx.at[s], ox_hbm.at[cs], semo[s]),
            pltpu.async_copy(oby.at[s], oy_hbm.at[cs], semo[s]),
            pltpu.async_copy(obz.at[s], oz_hbm.at[cs], semo[s]),
        ]

    def it1(s):
        @plsc.parallel_loop(0, _CN // 16, unroll=4)
        def body(i):
            sl = pl.ds(i * 16, 16)
            om = ombuf[s, sl]
            grow = gbuf[s, sl]
            pos = (om - om0) * inv_h
            c0 = jnp.clip(pos.astype(jnp.int32), 0, _N_OM - 2)
            w0 = plsc.load_gather(omg, [c0])
            w1 = plsc.load_gather(omg, [c0 + 1])
            j = c0 + (w0 < om).astype(jnp.int32) + (w1 < om).astype(jnp.int32)
            c = jnp.clip(j, 1, _N_OM - 1)
            x1 = plsc.load_gather(omg, [c])
            x0 = plsc.load_gather(omg, [c - 1])
            g0 = grow + c - 1
            tbuf[s, sl] = (om - x0) / (x1 - x0)
            ibuf[s, sl] = 1.0 / om
            g0buf[s][sl] = g0
            g1buf[s][sl] = g0 + 1

    def it2(s):
        @plsc.parallel_loop(0, _CN // 16, unroll=4)
        def body(i):
            sl = pl.ds(i * 16, 16)
            f0 = f0buf[s][sl]
            f1 = f1buf[s][sl]
            q = (f0 + (f1 - f0) * tbuf[s, sl]) * ibuf[s, sl]
            obx[s, sl] = q * xbuf[s, sl]
            oby[s, sl] = q * ybuf[s, sl]
            obz[s, sl] = q * zbuf[s, sl]

    in_h = [None, None]
    g_h = [None, None]
    o_h = [None, None]
    in_h[0] = start_in(0, 0)
    if nsub > 1:
        in_h[1] = start_in(1, 1)
    for k in range(nsub):
        s = k & 1
        s2 = 1 - s
        for h in in_h[s]:
            h.wait()
        it1(s)
        g_h[s] = [
            pltpu.async_copy(tab_hbm.at[g0buf[s]], f0buf[s], semg[s]),
            pltpu.async_copy(tab_hbm.at[g1buf[s]], f1buf[s], semg[s]),
        ]
        if k >= 1:
            for h in g_h[s2]:
                h.wait()
            if o_h[s2] is not None:
                for h in o_h[s2]:
                    h.wait()
            it2(s2)
            o_h[s2] = start_out(k - 1, s2)
            if k + 1 < nsub:
                in_h[s2] = start_in(k + 1, s2)
    s = (nsub - 1) & 1
    for h in g_h[s]:
        h.wait()
    if o_h[s] is not None:
        for h in o_h[s]:
            h.wait()
    it2(s)
    o_h[s] = start_out(nsub - 1, s)
    for ss in (0, 1):
        if o_h[ss] is not None:
            for h in o_h[ss]:
                h.wait()


def kernel(scale, vec, omegas_array, score_norms):
    b = scale.shape[0]
    xs = vec[:, 0]
    ys = vec[:, 1]
    zs = vec[:, 2]
    tab_flat = score_norms.reshape(_N_EPS * _N_OM)
    b2 = b // 2
    assert b2 % (_NW * _CN) == 0
    nsub = b2 // (_NW * _CN)
    grid = 8
    bs = b2 // grid
    outs = []
    for hh in range(2):
        sl = slice(hh * b2, (hh + 1) * b2)
        om, grow = pl.pallas_call(
            _tc_body,
            grid=(grid,),
            in_specs=[pl.BlockSpec((bs,), lambda i: (i,))] * 4,
            out_specs=[pl.BlockSpec((bs,), lambda i: (i,))] * 2,
            out_shape=[
                jax.ShapeDtypeStruct((b2,), jnp.float32),
                jax.ShapeDtypeStruct((b2,), jnp.int32),
            ],
        )(scale[sl], xs[sl], ys[sl], zs[sl])
        outs.append(_sc_main(nsub)(om, grow, xs[sl], ys[sl], zs[sl],
                                   omegas_array, tab_flat))
    ox = jnp.concatenate([outs[0][0], outs[1][0]])
    oy = jnp.concatenate([outs[0][1], outs[1][1]])
    oz = jnp.concatenate([outs[0][2], outs[1][2]])
    return jnp.stack([ox, oy, oz], axis=1).astype(scale.dtype)


def _sc_main(nsub):
    b = nsub * _NW * _CN
    h = (np.pi - 1e-3) / (_N_OM - 1)
    mesh = plsc.VectorSubcoreMesh(core_axis_name="c", subcore_axis_name="s")
    return pl.kernel(
        functools.partial(_sc_main_body, nsub=nsub,
                          inv_h=np.float32(1.0 / h), om0=np.float32(1e-3)),
        out_type=[jax.ShapeDtypeStruct((b,), jnp.float32)] * 3,
        mesh=mesh,
        scratch_types=[
            pltpu.VMEM((2, _CN), jnp.float32),
            pltpu.VMEM((2, _CN), jnp.int32),
            pltpu.VMEM((2, _CN), jnp.float32),
            pltpu.VMEM((2, _CN), jnp.float32),
            pltpu.VMEM((2, _CN), jnp.float32),
            pltpu.VMEM((2, _CN), jnp.float32),
            pltpu.VMEM((2, _CN), jnp.float32),
            pltpu.VMEM((_CN,), jnp.int32),
            pltpu.VMEM((_CN,), jnp.int32),
            pltpu.VMEM((_CN,), jnp.int32),
            pltpu.VMEM((_CN,), jnp.int32),
            pltpu.VMEM((_CN,), jnp.float32),
            pltpu.VMEM((_CN,), jnp.float32),
            pltpu.VMEM((_CN,), jnp.float32),
            pltpu.VMEM((_CN,), jnp.float32),
            pltpu.VMEM((2, _CN), jnp.float32),
            pltpu.VMEM((2, _CN), jnp.float32),
            pltpu.VMEM((2, _CN), jnp.float32),
            pltpu.VMEM((_N_OM,), jnp.float32),
            pltpu.SemaphoreType.DMA,
            pltpu.SemaphoreType.DMA,
            pltpu.SemaphoreType.DMA,
            pltpu.SemaphoreType.DMA,
            pltpu.SemaphoreType.DMA,
            pltpu.SemaphoreType.DMA,
        ],
        compiler_params=pltpu.CompilerParams(needs_layout_passes=False),
    )


# speedup vs baseline: 1.0723x; 1.0723x over previous
import functools
import numpy as np
import jax
import jax.numpy as jnp
from jax import lax
from jax.experimental import pallas as pl
from jax.experimental.pallas import tpu as pltpu
from jax.experimental.pallas import tpu_sc as plsc

_MIN_EPS = 0.01
_MAX_EPS = 2.0
_N_EPS = 1000
_N_OM = 1000

_NC, _NS = 2, 16
_NW = _NC * _NS
_CN = 4096


_C1 = np.float32(1.0 / np.log(10.0))
_C2 = np.float32(434.588)
_RNE_MAGIC = np.float32(12582912.0)


def _tc_body(scale_ref, x_ref, y_ref, z_ref, om_ref, grow_ref):
    eps = scale_ref[...]
    fi = (jnp.log(eps) * _C1 + np.float32(2.0)) * _C2
    r = (fi + _RNE_MAGIC) - _RNE_MAGIC
    ei = jnp.clip(r.astype(jnp.int32), 0, _N_EPS - 1)
    grow_ref[...] = ei * _N_OM
    x = x_ref[...]
    y = y_ref[...]
    z = z_ref[...]
    om_ref[...] = jnp.sqrt((x * x + z * z) + y * y)


def _sc_main_body(om_hbm, grow_hbm, x_hbm, y_hbm, z_hbm, omg_hbm, tab_hbm,
                  ox_hbm, oy_hbm, oz_hbm,
                  ombuf, gbuf, xbuf, ybuf, zbuf, tbuf, ibuf,
                  g0b0, g0b1, g1b0, g1b1, f0b0, f0b1, f1b0, f1b1, obx, oby, obz, omg,
                  semi0, semi1, semg0, semg1, semo0, semo1,
                  nsub, inv_h, om0):
    sid = lax.axis_index("s")
    wid = sid * _NC + lax.axis_index("c")
    base = wid * (nsub * _CN)
    semi = (semi0, semi1)
    semg = (semg0, semg1)
    semo = (semo0, semo1)
    g0buf = (g0b0, g0b1)
    g1buf = (g1b0, g1b1)
    f0buf = (f0b0, f0b1)
    f1buf = (f1b0, f1b1)
    pltpu.sync_copy(omg_hbm, omg)

    def start_in(k, s):
        cs = pl.ds(base + k * _CN, _CN)
        return [
            pltpu.async_copy(om_hbm.at[cs], ombuf.at[s], semi[s]),
            pltpu.async_copy(grow_hbm.at[cs], gbuf.at[s], semi[s]),
            pltpu.async_copy(x_hbm.at[cs], xbuf.at[s], semi[s]),
            pltpu.async_copy(y_hbm.at[cs], ybuf.at[s], semi[s]),
            pltpu.async_copy(z_hbm.at[cs], zbuf.at[s], semi[s]),
        ]

    def start_out(k, s):
        cs = pl.ds(base + k * _CN, _CN)
        return [
            pltpu.async_copy(obx.at[s], ox_hbm.at[cs], semo[s]),
            pltpu.async_copy(oby.at[s], oy_hbm.at[cs], semo[s]),
            pltpu.async_copy(obz.at[s], oz_hbm.at[cs], semo[s]),
        ]

    def it1(s):
        @plsc.parallel_loop(0, _CN // 16, unroll=4)
        def body(i):
            sl = pl.ds(i * 16, 16)
            om = ombuf[s, sl]
            grow = gbuf[s, sl]
            pos = (om - om0) * inv_h
            c0 = jnp.clip(pos.astype(jnp.int32), 0, _N_OM - 2)
            w0 = plsc.load_gather(omg, [c0])
            w1 = plsc.load_gather(omg, [c0 + 1])
            j = c0 + (w0 < om).astype(jnp.int32) + (w1 < om).astype(jnp.int32)
            c = jnp.clip(j, 1, _N_OM - 1)
            x1 = plsc.load_gather(omg, [c])
            x0 = plsc.load_gather(omg, [c - 1])
            g0 = grow + c - 1
            tbuf[s, sl] = (om - x0) / (x1 - x0)
            ibuf[s, sl] = 1.0 / om
            g0buf[s][sl] = g0
            g1buf[s][sl] = g0 + 1

    def it2(s):
        @plsc.parallel_loop(0, _CN // 16, unroll=4)
        def body(i):
            sl = pl.ds(i * 16, 16)
            f0 = f0buf[s][sl]
            f1 = f1buf[s][sl]
            q = (f0 + (f1 - f0) * tbuf[s, sl]) * ibuf[s, sl]
            obx[s, sl] = q * xbuf[s, sl]
            oby[s, sl] = q * ybuf[s, sl]
            obz[s, sl] = q * zbuf[s, sl]

    in_h = [None, None]
    g_h = [None, None]
    o_h = [None, None]
    in_h[0] = start_in(0, 0)
    if nsub > 1:
        in_h[1] = start_in(1, 1)
    for k in range(nsub):
        s = k & 1
        s2 = 1 - s
        for h in in_h[s]:
            h.wait()
        it1(s)
        g_h[s] = [
            pltpu.async_copy(tab_hbm.at[g0buf[s]], f0buf[s], semg[s]),
            pltpu.async_copy(tab_hbm.at[g1buf[s]], f1buf[s], semg[s]),
        ]
        if k >= 1:
            for h in g_h[s2]:
                h.wait()
            if o_h[s2] is not None:
                for h in o_h[s2]:
                    h.wait()
            it2(s2)
            o_h[s2] = start_out(k - 1, s2)
            if k + 1 < nsub:
                in_h[s2] = start_in(k + 1, s2)
    s = (nsub - 1) & 1
    for h in g_h[s]:
        h.wait()
    if o_h[s] is not None:
        for h in o_h[s]:
            h.wait()
    it2(s)
    o_h[s] = start_out(nsub - 1, s)
    for ss in (0, 1):
        if o_h[ss] is not None:
            for h in o_h[ss]:
                h.wait()


def kernel(scale, vec, omegas_array, score_norms):
    b = scale.shape[0]
    assert b % (_NW * _CN) == 0
    nsub = b // (_NW * _CN)
    xs = vec[:, 0]
    ys = vec[:, 1]
    zs = vec[:, 2]
    tab_flat = score_norms.reshape(_N_EPS * _N_OM)

    grid = 16
    bs = b // grid
    om, grow = pl.pallas_call(
        _tc_body,
        grid=(grid,),
        in_specs=[pl.BlockSpec((bs,), lambda i: (i,))] * 4,
        out_specs=[pl.BlockSpec((bs,), lambda i: (i,))] * 2,
        out_shape=[
            jax.ShapeDtypeStruct((b,), jnp.float32),
            jax.ShapeDtypeStruct((b,), jnp.int32),
        ],
    )(scale, xs, ys, zs)

    h = (np.pi - 1e-3) / (_N_OM - 1)
    mesh = plsc.VectorSubcoreMesh(core_axis_name="c", subcore_axis_name="s")
    main = pl.kernel(
        functools.partial(_sc_main_body, nsub=nsub,
                          inv_h=np.float32(1.0 / h), om0=np.float32(1e-3)),
        out_type=[jax.ShapeDtypeStruct((b,), jnp.float32)] * 3,
        mesh=mesh,
        scratch_types=[
            pltpu.VMEM((2, _CN), jnp.float32),
            pltpu.VMEM((2, _CN), jnp.int32),
            pltpu.VMEM((2, _CN), jnp.float32),
            pltpu.VMEM((2, _CN), jnp.float32),
            pltpu.VMEM((2, _CN), jnp.float32),
            pltpu.VMEM((2, _CN), jnp.float32),
            pltpu.VMEM((2, _CN), jnp.float32),
            pltpu.VMEM((_CN,), jnp.int32),
            pltpu.VMEM((_CN,), jnp.int32),
            pltpu.VMEM((_CN,), jnp.int32),
            pltpu.VMEM((_CN,), jnp.int32),
            pltpu.VMEM((_CN,), jnp.float32),
            pltpu.VMEM((_CN,), jnp.float32),
            pltpu.VMEM((_CN,), jnp.float32),
            pltpu.VMEM((_CN,), jnp.float32),
            pltpu.VMEM((2, _CN), jnp.float32),
            pltpu.VMEM((2, _CN), jnp.float32),
            pltpu.VMEM((2, _CN), jnp.float32),
            pltpu.VMEM((_N_OM,), jnp.float32),
            pltpu.SemaphoreType.DMA,
            pltpu.SemaphoreType.DMA,
            pltpu.SemaphoreType.DMA,
            pltpu.SemaphoreType.DMA,
            pltpu.SemaphoreType.DMA,
            pltpu.SemaphoreType.DMA,
        ],
        compiler_params=pltpu.CompilerParams(needs_layout_passes=False),
    )
    ox, oy, oz = main(om, grow, xs, ys, zs, omegas_array, tab_flat)
    return jnp.stack([ox, oy, oz], axis=1).astype(scale.dtype)
